# baseline (speedup 1.0000x reference)
import functools

import jax
import jax.numpy as jnp
from jax import lax
from jax.experimental import pallas as pl
from jax.experimental.pallas import tpu as pltpu
from jax.experimental.pallas import tpu_sc as plsc

B = 16384
D = 32
H1 = 128
H2 = 64
V = 1000000

_info = plsc.get_sparse_core_info()
_NC, _NS = _info.num_cores, _info.num_subcores
_NW = _NC * _NS
_CW = 512
_CPW = 61
_TAIL0 = _CW * (_CPW * _NW + 1)
_NG = B // 16

_mesh = plsc.VectorSubcoreMesh(core_axis_name="c", subcore_axis_name="s")


def _scan_one_chunk(ic, cnt, idx_chunk_v, idx_hbm, sem, c0, c1,
                    match_idx_v, match_pos_v):
    iota = lax.iota(jnp.int32, 16)
    pltpu.async_copy(
        idx_hbm.at[pl.ds(ic * 1024, 1024)], idx_chunk_v, sem).wait()

    def scan_group(g, cnt):
        vec = idx_chunk_v[pl.ds(g * 16, 16)]
        mask = (vec >= c0) & (vec < c1)
        cum = plsc.cumsum(jnp.where(mask, 1, 0).astype(jnp.int32))
        dst = cnt + cum - 1
        plsc.store_scatter(match_idx_v, [dst], vec, mask=mask)
        pos = iota + (ic * 1024 + g * 16)
        plsc.store_scatter(match_pos_v, [dst], pos, mask=mask)
        return cnt + cum[15]

    return lax.fori_loop(0, 64, scan_group, cnt, unroll=4)


def _scan_indices(idx_chunk_v, idx_hbm, sem, c0, c1, match_idx_v, match_pos_v):
    def scan_chunk(ic, cnt):
        return _scan_one_chunk(ic, cnt, idx_chunk_v, idx_hbm, sem, c0, c1,
                               match_idx_v, match_pos_v)

    return lax.fori_loop(0, B // 1024, scan_chunk, jnp.int32(0))


def _gather_phase(tabT_hbm, tail_hbm, out_hbm, bufs, sems, row_sem,
                  match_idx_v, match_pos_v,
                  m2_v, rbase_v, rowgrp_v, c0, c1, nch, wid,
                  cnt, background=None):
    iota = lax.iota(jnp.int32, 16)
    ngrp = (cnt + 15) // 16

    def issue_chunk(c, buf, sem):
        @pl.when(c < nch)
        def _():
            lo = pl.multiple_of(c0 + c * _CW, 128)
            pltpu.async_copy(tabT_hbm.at[:, pl.ds(lo, _CW)], buf, sem)

    def wait_chunk(buf, sem):
        pltpu.make_async_copy(tabT_hbm.at[:, pl.ds(0, _CW)], buf, sem).wait()

    def drain_rows(n):
        def w(_, x):
            pltpu.make_async_copy(rowgrp_v.at[pl.ds(0, 1)],
                                  out_hbm.at[pl.ds(0, 1)], row_sem).wait()
            return x
        lax.fori_loop(0, n, w, jnp.int32(0))

    def pass_r(r, carry):
        cur0, bases = carry
        bases = jnp.where(iota == r, cur0, bases)

        def grp(g, cur):
            mvec = match_idx_v[pl.ds(g * 16, 16)]
            pvec = match_pos_v[pl.ds(g * 16, 16)]
            valid = (iota + g * 16) < cnt
            rid = lax.shift_right_logical(mvec - c0, 12)
            m = valid & (rid == r)
            cum = plsc.cumsum(jnp.where(m, 1, 0).astype(jnp.int32))
            dst = cur + cum - 1
            packed = jnp.bitwise_or(
                lax.shift_left(mvec - c0, 14), pvec)
            plsc.store_scatter(m2_v, [dst], packed, mask=m)
            return cur + cum[15]

        cur1 = lax.fori_loop(0, ngrp, grp, cur0)
        return (cur1, bases)

    tot, bases = lax.fori_loop(
        0, 8, pass_r, (jnp.int32(0), jnp.zeros((16,), jnp.int32)))
    bases = jnp.where(iota >= 8, tot, bases)
    rbase_v[pl.ds(0, 16)] = bases

    def process_chunk(c, buf, e0):
        lo = c0 + c * _CW

        plo = lax.shift_left(lo - c0, 14)
        phi = lax.shift_left(lo - c0 + _CW, 14)

        def group_walk(g, e):
            pval = m2_v[pl.ds(g * 16, 16)]
            valid = (iota + g * 16) < cnt
            m = (pval >= plo) & (pval < phi) & valid

            def group_body(e):
                pc = plsc.all_reduce_population_count(m)[0]
                cum = plsc.cumsum(jnp.where(m, 1, 0).astype(jnp.int32))

                def hit(r, e):
                    sel = m & (cum == r + 1)
                    p = jnp.sum(jnp.where(sel, pval, 0))
                    j = lax.shift_right_logical(p, 14) - (lo - c0)
                    b = jnp.bitwise_and(p, 16383)
                    slot = lax.rem(e, jnp.int32(64))
                    jv = jnp.full((16,), j, jnp.int32)
                    r0 = plsc.load_gather(buf, [iota, jv])
                    r1 = plsc.load_gather(buf, [iota + 16, jv])
                    srow = jnp.full((16,), slot, jnp.int32)
                    plsc.store_scatter(rowgrp_v, [srow, iota], r0)
                    plsc.store_scatter(rowgrp_v, [srow, iota + 16], r1)
                    pltpu.async_copy(rowgrp_v.at[pl.ds(slot, 1)],
                                     out_hbm.at[pl.ds(b, 1)], row_sem)

                    @pl.when(slot == 63)
                    def _():
                        drain_rows(64)
                    return e + 1

                return lax.fori_loop(0, pc, hit, e)

            return lax.cond(jnp.any(m), group_body, lambda e: e, e)

        rc = lax.shift_right_logical(c, 3)
        b0 = plsc.load_gather(rbase_v, [jnp.full((16,), rc, jnp.int32)])[0]
        b1 = plsc.load_gather(rbase_v, [jnp.full((16,), rc + 1,
                                                 jnp.int32)])[0]
        return lax.fori_loop(lax.shift_right_logical(b0, 4),
                             lax.shift_right_logical(b1 + 15, 4),
                             group_walk, e0)

    nb = len(bufs)
    for i in range(nb):
        issue_chunk(jnp.int32(i), bufs[i], sems[i])

    def ring(t, carry):
        e, bg = carry
        for i in range(nb):
            c = nb * t + i

            @pl.when(c < nch)
            def _(i=i):
                wait_chunk(bufs[i], sems[i])
            e = lax.cond(
                c < nch,
                lambda e, c=c, i=i: process_chunk(c, bufs[i], e),
                lambda e: e, e)
            issue_chunk(c + nb, bufs[i], sems[i])
        if background is not None:
            bg = background(t, bg)
        return (e, bg)

    e, bg_out = lax.fori_loop(0, (_CPW + 1 + nb - 1) // nb, ring,
                              (jnp.int32(0), jnp.int32(0)))
    drain_rows(lax.rem(e, jnp.int32(64)))

    ret = bg_out

    @pl.when(wid == _NW - 1)
    def _():
        def tail_walk(g, t):
            mvec = match_idx_v[pl.ds(g * 16, 16)]
            pvec = match_pos_v[pl.ds(g * 16, 16)]
            for k in range(16):
                jk = mvec[k]

                @pl.when((jk >= _TAIL0) & (g * 16 + k < cnt))
                def _():
                    pltpu.async_copy(tail_hbm.at[pl.ds(jk - _TAIL0, 1)],
                                     rowgrp_v.at[pl.ds(0, 1)], row_sem).wait()
                    pltpu.async_copy(rowgrp_v.at[pl.ds(0, 1)],
                                     out_hbm.at[pl.ds(pvec[k], 1)],
                                     row_sem).wait()
            return t

        lax.fori_loop(0, ngrp, tail_walk, jnp.int32(0))

    return ret


@functools.partial(
    pl.kernel,
    mesh=_mesh,
    compiler_params=pltpu.CompilerParams(needs_layout_passes=False),
    out_type=[
        jax.ShapeDtypeStruct((B, D), jnp.float32),
        jax.ShapeDtypeStruct((B, D), jnp.float32),
    ],
    scratch_types=[
        pltpu.VMEM((D, _CW), jnp.float32),
        pltpu.VMEM((D, _CW), jnp.float32),
        pltpu.VMEM((1024,), jnp.int32),
        pltpu.VMEM((B + 16,), jnp.int32),
        pltpu.VMEM((B + 16,), jnp.int32),
        pltpu.VMEM((B + 16,), jnp.int32),
        pltpu.VMEM((B + 16,), jnp.int32),
        pltpu.VMEM((B + 16,), jnp.int32),
        pltpu.VMEM((16,), jnp.int32),
        pltpu.VMEM((64, D), jnp.float32),
        pltpu.SemaphoreType.DMA,
        pltpu.SemaphoreType.DMA,
        pltpu.SemaphoreType.DMA,
        pltpu.SemaphoreType.DMA,
    ],
)
def _sc_gather(uidx_hbm, iidx_hbm, utabT_hbm, itabT_hbm, utail_hbm, itail_hbm,
               uout_hbm, iout_hbm,
               buf0, buf1, idx_chunk_v, match_u_idx_v, match_u_pos_v,
               match_i_idx_v, match_i_pos_v,
               m2_v, rbase_v, rowgrp_v,
               sem0, sem1, idx_sem, row_sem):
    wid = lax.axis_index("s") * _NC + lax.axis_index("c")
    c0 = wid * (_CPW * _CW)
    last = wid == _NW - 1
    nch = jnp.where(last, _CPW + 1, _CPW)
    c1 = jnp.where(last, V, c0 + _CPW * _CW)
    cnt_u = _scan_indices(idx_chunk_v, uidx_hbm, idx_sem, c0, c1,
                          match_u_idx_v, match_u_pos_v)

    def scan_item_slice(t, ci):
        return lax.cond(
            t < B // 1024,
            lambda ci: _scan_one_chunk(t, ci, idx_chunk_v, iidx_hbm, idx_sem,
                                       c0, c1, match_i_idx_v, match_i_pos_v),
            lambda ci: ci, ci)

    cnt_i = _gather_phase(utabT_hbm, utail_hbm, uout_hbm, (buf0, buf1),
                          (sem0, sem1), row_sem,
                          match_u_idx_v, match_u_pos_v, m2_v,
                          rbase_v, rowgrp_v, c0, c1, nch, wid, cnt_u,
                          background=scan_item_slice)
    _gather_phase(itabT_hbm, itail_hbm, iout_hbm, (buf0, buf1),
                  (sem0, sem1), row_sem,
                  match_i_idx_v, match_i_pos_v, m2_v,
                  rbase_v, rowgrp_v, c0, c1, nch, wid, cnt_i)


_BLK = 8192


def _mlp_body(uvec_ref, ivec_ref, w1u_ref, w1i_ref, b1_ref, w2_ref, b2_ref,
              w3_ref, b3_ref, out_ref):
    h = jnp.dot(uvec_ref[...], w1u_ref[...], preferred_element_type=jnp.float32)
    h += jnp.dot(ivec_ref[...], w1i_ref[...], preferred_element_type=jnp.float32)
    h = jnp.maximum(h + b1_ref[...], 0.0)
    h = jnp.dot(h, w2_ref[...], preferred_element_type=jnp.float32)
    h = jnp.maximum(h + b2_ref[...], 0.0)
    out_ref[...] = (
        jnp.dot(h, w3_ref[...], preferred_element_type=jnp.float32)
        + b3_ref[...]
    )


def _mlp(uvec, ivec, W1u, W1i, b1, W2, b2, W3, b3):
    grid = (B // _BLK,)
    return pl.pallas_call(
        _mlp_body,
        grid=grid,
        in_specs=[
            pl.BlockSpec((_BLK, D), lambda i: (i, 0)),
            pl.BlockSpec((_BLK, D), lambda i: (i, 0)),
            pl.BlockSpec((D, H1), lambda i: (0, 0)),
            pl.BlockSpec((D, H1), lambda i: (0, 0)),
            pl.BlockSpec((1, H1), lambda i: (0, 0)),
            pl.BlockSpec((H1, H2), lambda i: (0, 0)),
            pl.BlockSpec((1, H2), lambda i: (0, 0)),
            pl.BlockSpec((H2, 1), lambda i: (0, 0)),
            pl.BlockSpec((1, 1), lambda i: (0, 0)),
        ],
        out_specs=pl.BlockSpec((_BLK, 1), lambda i: (i, 0)),
        out_shape=jax.ShapeDtypeStruct((B, 1), jnp.float32),
    )(uvec, ivec, W1u, W1i, b1, W2, b2, W3, b3)


def kernel(user_idx, item_idx, user_table, item_table, W1, b1, W2, b2, W3, b3):
    user_idx = user_idx.astype(jnp.int32)
    item_idx = item_idx.astype(jnp.int32)
    uvec, ivec = _sc_gather(
        user_idx, item_idx, user_table.T, item_table.T,
        user_table[_TAIL0:], item_table[_TAIL0:])
    out = _mlp(
        uvec, ivec,
        W1[:D], W1[D:], b1.reshape(1, H1),
        W2, b2.reshape(1, H2),
        W3, b3.reshape(1, 1),
    )
    return jnp.squeeze(out, axis=-1)


# speedup vs baseline: 1.0645x; 1.0645x over previous
import functools

import jax
import jax.numpy as jnp
from jax import lax
from jax.experimental import pallas as pl
from jax.experimental.pallas import tpu as pltpu
from jax.experimental.pallas import tpu_sc as plsc

B = 16384
D = 32
H1 = 128
H2 = 64
V = 1000000

_info = plsc.get_sparse_core_info()
_NC, _NS = _info.num_cores, _info.num_subcores
_NW = _NC * _NS
_CW = 512
_CPW = 61
_TAIL0 = _CW * (_CPW * _NW + 1)
_NG = B // 16

_mesh = plsc.VectorSubcoreMesh(core_axis_name="c", subcore_axis_name="s")


def _scan_one_chunk(ic, cnt, idx_chunk_v, idx_hbm, sem, c0, c1, match_v):
    iota = lax.iota(jnp.int32, 16)
    pltpu.async_copy(
        idx_hbm.at[pl.ds(ic * 1024, 1024)], idx_chunk_v, sem).wait()

    def scan_group(g, cnt):
        vec = idx_chunk_v[pl.ds(g * 16, 16)]
        mask = (vec >= c0) & (vec < c1)
        cum = plsc.cumsum(jnp.where(mask, 1, 0).astype(jnp.int32))
        dst = cnt + cum - 1
        pos = iota + (ic * 1024 + g * 16)
        packed = jnp.bitwise_or(lax.shift_left(vec - c0, 14), pos)
        plsc.store_scatter(match_v, [dst], packed, mask=mask)
        return cnt + cum[15]

    return lax.fori_loop(0, 64, scan_group, cnt, unroll=4)


def _scan_indices(idx_chunk_v, idx_hbm, sem, c0, c1, match_v):
    def scan_chunk(ic, cnt):
        return _scan_one_chunk(ic, cnt, idx_chunk_v, idx_hbm, sem, c0, c1,
                               match_v)

    return lax.fori_loop(0, B // 1024, scan_chunk, jnp.int32(0))


def _gather_phase(tabT_hbm, tail_hbm, out_hbm, bufs, sems, row_sem,
                  match_v,
                  m2_v, rbase_v, rowgrp_v, c0, c1, nch, wid,
                  cnt, background=None):
    iota = lax.iota(jnp.int32, 16)
    ngrp = (cnt + 15) // 16

    def issue_chunk(c, buf, sem):
        @pl.when(c < nch)
        def _():
            lo = pl.multiple_of(c0 + c * _CW, 128)
            pltpu.async_copy(tabT_hbm.at[:, pl.ds(lo, _CW)], buf, sem)

    def wait_chunk(buf, sem):
        pltpu.make_async_copy(tabT_hbm.at[:, pl.ds(0, _CW)], buf, sem).wait()

    def drain_rows(n):
        def w(_, x):
            pltpu.make_async_copy(rowgrp_v.at[pl.ds(0, 1)],
                                  out_hbm.at[pl.ds(0, 1)], row_sem).wait()
            return x
        lax.fori_loop(0, n, w, jnp.int32(0))

    def pass_r(r, carry):
        cur0, bases = carry
        bases = jnp.where(iota == r, cur0, bases)

        def grp(g, cur):
            pval = match_v[pl.ds(g * 16, 16)]
            valid = (iota + g * 16) < cnt
            rid = lax.shift_right_logical(pval, 26)
            m = valid & (rid == r)
            cum = plsc.cumsum(jnp.where(m, 1, 0).astype(jnp.int32))
            dst = cur + cum - 1
            plsc.store_scatter(m2_v, [dst], pval, mask=m)
            return cur + cum[15]

        cur1 = lax.fori_loop(0, ngrp, grp, cur0)
        return (cur1, bases)

    tot, bases = lax.fori_loop(
        0, 8, pass_r, (jnp.int32(0), jnp.zeros((16,), jnp.int32)))
    bases = jnp.where(iota >= 8, tot, bases)
    rbase_v[pl.ds(0, 16)] = bases

    def process_chunk(c, buf, e0):
        lo = c0 + c * _CW

        plo = lax.shift_left(lo - c0, 14)
        phi = lax.shift_left(lo - c0 + _CW, 14)

        def group_walk(g, e):
            pval = m2_v[pl.ds(g * 16, 16)]
            valid = (iota + g * 16) < cnt
            m = (pval >= plo) & (pval < phi) & valid

            def group_body(e):
                pc = plsc.all_reduce_population_count(m)[0]
                cum = plsc.cumsum(jnp.where(m, 1, 0).astype(jnp.int32))

                def hit(r, e):
                    sel = m & (cum == r + 1)
                    p = jnp.sum(jnp.where(sel, pval, 0))
                    j = lax.shift_right_logical(p, 14) - (lo - c0)
                    b = jnp.bitwise_and(p, 16383)
                    slot = lax.rem(e, jnp.int32(64))
                    jv = jnp.full((16,), j, jnp.int32)
                    r0 = plsc.load_gather(buf, [iota, jv])
                    r1 = plsc.load_gather(buf, [iota + 16, jv])
                    srow = jnp.full((16,), slot, jnp.int32)
                    plsc.store_scatter(rowgrp_v, [srow, iota], r0)
                    plsc.store_scatter(rowgrp_v, [srow, iota + 16], r1)
                    pltpu.async_copy(rowgrp_v.at[pl.ds(slot, 1)],
                                     out_hbm.at[pl.ds(b, 1)], row_sem)

                    @pl.when(slot == 63)
                    def _():
                        drain_rows(64)
                    return e + 1

                return lax.fori_loop(0, pc, hit, e)

            return lax.cond(jnp.any(m), group_body, lambda e: e, e)

        rc = lax.shift_right_logical(c, 3)
        b0 = plsc.load_gather(rbase_v, [jnp.full((16,), rc, jnp.int32)])[0]
        b1 = plsc.load_gather(rbase_v, [jnp.full((16,), rc + 1,
                                                 jnp.int32)])[0]
        return lax.fori_loop(lax.shift_right_logical(b0, 4),
                             lax.shift_right_logical(b1 + 15, 4),
                             group_walk, e0)

    nb = len(bufs)
    for i in range(nb):
        issue_chunk(jnp.int32(i), bufs[i], sems[i])

    def ring(t, carry):
        e, bg = carry
        for i in range(nb):
            c = nb * t + i

            @pl.when(c < nch)
            def _(i=i):
                wait_chunk(bufs[i], sems[i])
            e = lax.cond(
                c < nch,
                lambda e, c=c, i=i: process_chunk(c, bufs[i], e),
                lambda e: e, e)
            issue_chunk(c + nb, bufs[i], sems[i])
        if background is not None:
            bg = background(t, bg)
        return (e, bg)

    e, bg_out = lax.fori_loop(0, (_CPW + 1 + nb - 1) // nb, ring,
                              (jnp.int32(0), jnp.int32(0)))
    drain_rows(lax.rem(e, jnp.int32(64)))

    ret = bg_out

    @pl.when(wid == _NW - 1)
    def _():
        def tail_walk(g, t):
            pvalv = match_v[pl.ds(g * 16, 16)]
            for k in range(16):
                p = pvalv[k]
                jk = lax.shift_right_logical(p, 14) + c0
                bk = jnp.bitwise_and(p, 16383)

                @pl.when((jk >= _TAIL0) & (g * 16 + k < cnt))
                def _():
                    pltpu.async_copy(tail_hbm.at[pl.ds(jk - _TAIL0, 1)],
                                     rowgrp_v.at[pl.ds(0, 1)], row_sem).wait()
                    pltpu.async_copy(rowgrp_v.at[pl.ds(0, 1)],
                                     out_hbm.at[pl.ds(bk, 1)],
                                     row_sem).wait()
            return t

        lax.fori_loop(0, ngrp, tail_walk, jnp.int32(0))

    return ret


@functools.partial(
    pl.kernel,
    mesh=_mesh,
    compiler_params=pltpu.CompilerParams(needs_layout_passes=False),
    out_type=[
        jax.ShapeDtypeStruct((B, D), jnp.float32),
        jax.ShapeDtypeStruct((B, D), jnp.float32),
    ],
    scratch_types=[
        pltpu.VMEM((D, _CW), jnp.float32),
        pltpu.VMEM((D, _CW), jnp.float32),
        pltpu.VMEM((D, _CW), jnp.float32),
        pltpu.VMEM((1024,), jnp.int32),
        pltpu.VMEM((B + 16,), jnp.int32),
        pltpu.VMEM((B + 16,), jnp.int32),
        pltpu.VMEM((B + 16,), jnp.int32),
        pltpu.VMEM((16,), jnp.int32),
        pltpu.VMEM((64, D), jnp.float32),
        pltpu.SemaphoreType.DMA,
        pltpu.SemaphoreType.DMA,
        pltpu.SemaphoreType.DMA,
        pltpu.SemaphoreType.DMA,
        pltpu.SemaphoreType.DMA,
    ],
)
def _sc_gather(uidx_hbm, iidx_hbm, utabT_hbm, itabT_hbm, utail_hbm, itail_hbm,
               uout_hbm, iout_hbm,
               buf0, buf1, buf2, idx_chunk_v, match_u_v, match_i_v,
               m2_v, rbase_v, rowgrp_v,
               sem0, sem1, sem2, idx_sem, row_sem):
    wid = lax.axis_index("s") * _NC + lax.axis_index("c")
    c0 = wid * (_CPW * _CW)
    last = wid == _NW - 1
    nch = jnp.where(last, _CPW + 1, _CPW)
    c1 = jnp.where(last, V, c0 + _CPW * _CW)
    cnt_u = _scan_indices(idx_chunk_v, uidx_hbm, idx_sem, c0, c1,
                          match_u_v)

    def scan_item_slice(t, ci):
        return lax.cond(
            t < B // 1024,
            lambda ci: _scan_one_chunk(t, ci, idx_chunk_v, iidx_hbm, idx_sem,
                                       c0, c1, match_i_v),
            lambda ci: ci, ci)

    cnt_i = _gather_phase(utabT_hbm, utail_hbm, uout_hbm, (buf0, buf1, buf2),
                          (sem0, sem1, sem2), row_sem,
                          match_u_v, m2_v,
                          rbase_v, rowgrp_v, c0, c1, nch, wid, cnt_u,
                          background=scan_item_slice)
    _gather_phase(itabT_hbm, itail_hbm, iout_hbm, (buf0, buf1, buf2),
                  (sem0, sem1, sem2), row_sem,
                  match_i_v, m2_v,
                  rbase_v, rowgrp_v, c0, c1, nch, wid, cnt_i)


_BLK = 8192


def _mlp_body(uvec_ref, ivec_ref, w1u_ref, w1i_ref, b1_ref, w2_ref, b2_ref,
              w3_ref, b3_ref, out_ref):
    h = jnp.dot(uvec_ref[...], w1u_ref[...], preferred_element_type=jnp.float32)
    h += jnp.dot(ivec_ref[...], w1i_ref[...], preferred_element_type=jnp.float32)
    h = jnp.maximum(h + b1_ref[...], 0.0)
    h = jnp.dot(h, w2_ref[...], preferred_element_type=jnp.float32)
    h = jnp.maximum(h + b2_ref[...], 0.0)
    out_ref[...] = (
        jnp.dot(h, w3_ref[...], preferred_element_type=jnp.float32)
        + b3_ref[...]
    )


def _mlp(uvec, ivec, W1u, W1i, b1, W2, b2, W3, b3):
    grid = (B // _BLK,)
    return pl.pallas_call(
        _mlp_body,
        grid=grid,
        in_specs=[
            pl.BlockSpec((_BLK, D), lambda i: (i, 0)),
            pl.BlockSpec((_BLK, D), lambda i: (i, 0)),
            pl.BlockSpec((D, H1), lambda i: (0, 0)),
            pl.BlockSpec((D, H1), lambda i: (0, 0)),
            pl.BlockSpec((1, H1), lambda i: (0, 0)),
            pl.BlockSpec((H1, H2), lambda i: (0, 0)),
            pl.BlockSpec((1, H2), lambda i: (0, 0)),
            pl.BlockSpec((H2, 1), lambda i: (0, 0)),
            pl.BlockSpec((1, 1), lambda i: (0, 0)),
        ],
        out_specs=pl.BlockSpec((_BLK, 1), lambda i: (i, 0)),
        out_shape=jax.ShapeDtypeStruct((B, 1), jnp.float32),
    )(uvec, ivec, W1u, W1i, b1, W2, b2, W3, b3)


def kernel(user_idx, item_idx, user_table, item_table, W1, b1, W2, b2, W3, b3):
    user_idx = user_idx.astype(jnp.int32)
    item_idx = item_idx.astype(jnp.int32)
    uvec, ivec = _sc_gather(
        user_idx, item_idx, user_table.T, item_table.T,
        user_table[_TAIL0:], item_table[_TAIL0:])
    out = _mlp(
        uvec, ivec,
        W1[:D], W1[D:], b1.reshape(1, H1),
        W2, b2.reshape(1, H2),
        W3, b3.reshape(1, 1),
    )
    return jnp.squeeze(out, axis=-1)


# speedup vs baseline: 1.0832x; 1.0176x over previous
import functools

import jax
import jax.numpy as jnp
from jax import lax
from jax.experimental import pallas as pl
from jax.experimental.pallas import tpu as pltpu
from jax.experimental.pallas import tpu_sc as plsc

B = 16384
D = 32
H1 = 128
H2 = 64
V = 1000000

_info = plsc.get_sparse_core_info()
_NC, _NS = _info.num_cores, _info.num_subcores
_NW = _NC * _NS
_CW = 512
_CPW = 61
_TAIL0 = _CW * (_CPW * _NW + 1)
_NG = B // 16

_mesh = plsc.VectorSubcoreMesh(core_axis_name="c", subcore_axis_name="s")


def _scan_one_chunk(ic, cnt, idx_chunk_v, idx_hbm, sem, c0, c1, match_v):
    iota = lax.iota(jnp.int32, 16)
    pltpu.async_copy(
        idx_hbm.at[pl.ds(ic * 1024, 1024)], idx_chunk_v, sem).wait()

    def scan_group(g, cnt):
        vec = idx_chunk_v[pl.ds(g * 16, 16)]
        mask = (vec >= c0) & (vec < c1)
        cum = plsc.cumsum(jnp.where(mask, 1, 0).astype(jnp.int32))
        dst = cnt + cum - 1
        pos = iota + (ic * 1024 + g * 16)
        packed = jnp.bitwise_or(lax.shift_left(vec - c0, 14), pos)
        plsc.store_scatter(match_v, [dst], packed, mask=mask)
        return cnt + cum[15]

    return lax.fori_loop(0, 64, scan_group, cnt, unroll=4)


def _scan_indices(idx_chunk_v, idx_hbm, sem, c0, c1, match_v):
    def scan_chunk(ic, cnt):
        return _scan_one_chunk(ic, cnt, idx_chunk_v, idx_hbm, sem, c0, c1,
                               match_v)

    return lax.fori_loop(0, B // 1024, scan_chunk, jnp.int32(0))


def _gather_phase(tabT_hbm, tail_hbm, out_hbm, bufs, sems, row_sem,
                  match_v,
                  m2_v, rbase_v, rowgrp_v, c0, c1, nch, wid,
                  cnt, background=None):
    iota = lax.iota(jnp.int32, 16)
    ngrp = (cnt + 15) // 16

    def issue_chunk(c, buf, sem):
        @pl.when(c < nch)
        def _():
            lo = pl.multiple_of(c0 + c * _CW, 128)
            pltpu.async_copy(tabT_hbm.at[:, pl.ds(lo, _CW)], buf, sem)

    def wait_chunk(buf, sem):
        pltpu.make_async_copy(tabT_hbm.at[:, pl.ds(0, _CW)], buf, sem).wait()

    def drain_rows(n):
        def w(_, x):
            pltpu.make_async_copy(rowgrp_v.at[pl.ds(0, 1)],
                                  out_hbm.at[pl.ds(0, 1)], row_sem).wait()
            return x
        lax.fori_loop(0, n, w, jnp.int32(0))

    def pass_r(r, carry):
        cur0, bases = carry
        bases = jnp.where(iota == r, cur0, bases)

        def grp(g, cur):
            pval = match_v[pl.ds(g * 16, 16)]
            valid = (iota + g * 16) < cnt
            rid = lax.shift_right_logical(pval, 26)
            m = valid & (rid == r)
            cum = plsc.cumsum(jnp.where(m, 1, 0).astype(jnp.int32))
            dst = cur + cum - 1
            plsc.store_scatter(m2_v, [dst], pval, mask=m)
            return cur + cum[15]

        cur1 = lax.fori_loop(0, ngrp, grp, cur0)
        return (cur1, bases)

    tot, bases = lax.fori_loop(
        0, 8, pass_r, (jnp.int32(0), jnp.zeros((16,), jnp.int32)))
    bases = jnp.where(iota >= 8, tot, bases)
    rbase_v[pl.ds(0, 16)] = bases

    def process_chunk(c, buf, e0):
        lo = c0 + c * _CW

        plo = lax.shift_left(lo - c0, 14)
        phi = lax.shift_left(lo - c0 + _CW, 14)

        def group_walk(g, e):
            pval = m2_v[pl.ds(g * 16, 16)]
            valid = (iota + g * 16) < cnt
            m = (pval >= plo) & (pval < phi) & valid

            def group_body(e):
                pc = plsc.all_reduce_population_count(m)[0]
                cum = plsc.cumsum(jnp.where(m, 1, 0).astype(jnp.int32))

                def hit(r, e):
                    sel = m & (cum == r + 1)
                    p = jnp.sum(jnp.where(sel, pval, 0))
                    j = lax.shift_right_logical(p, 14) - (lo - c0)
                    b = jnp.bitwise_and(p, 16383)
                    slot = lax.rem(e, jnp.int32(64))
                    jv = jnp.full((16,), j, jnp.int32)
                    r0 = plsc.load_gather(buf, [iota, jv])
                    r1 = plsc.load_gather(buf, [iota + 16, jv])
                    srow = jnp.full((16,), slot, jnp.int32)
                    plsc.store_scatter(rowgrp_v, [srow, iota], r0)
                    plsc.store_scatter(rowgrp_v, [srow, iota + 16], r1)
                    pltpu.async_copy(rowgrp_v.at[pl.ds(slot, 1)],
                                     out_hbm.at[pl.ds(b, 1)], row_sem)

                    @pl.when(slot == 63)
                    def _():
                        drain_rows(64)
                    return e + 1

                return lax.fori_loop(0, pc, hit, e)

            return lax.cond(jnp.any(m), group_body, lambda e: e, e)

        rc = lax.shift_right_logical(c, 3)
        b0 = plsc.load_gather(rbase_v, [jnp.full((16,), rc, jnp.int32)])[0]
        b1 = plsc.load_gather(rbase_v, [jnp.full((16,), rc + 1,
                                                 jnp.int32)])[0]
        return lax.fori_loop(lax.shift_right_logical(b0, 4),
                             lax.shift_right_logical(b1 + 15, 4),
                             group_walk, e0)

    nb = len(bufs)
    for i in range(nb):
        issue_chunk(jnp.int32(i), bufs[i], sems[i])

    def ring(t, carry):
        e, bg = carry
        for i in range(nb):
            c = nb * t + i

            @pl.when(c < nch)
            def _(i=i):
                wait_chunk(bufs[i], sems[i])
            e = lax.cond(
                c < nch,
                lambda e, c=c, i=i: process_chunk(c, bufs[i], e),
                lambda e: e, e)
            issue_chunk(c + nb, bufs[i], sems[i])
        if background is not None:
            bg = background(t, bg)
        return (e, bg)

    e, bg_out = lax.fori_loop(0, (_CPW + 1 + nb - 1) // nb, ring,
                              (jnp.int32(0), jnp.int32(0)))
    drain_rows(lax.rem(e, jnp.int32(64)))

    ret = bg_out

    @pl.when(wid == _NW - 1)
    def _():
        def tail_walk(g, t):
            pvalv = match_v[pl.ds(g * 16, 16)]
            for k in range(16):
                p = pvalv[k]
                jk = lax.shift_right_logical(p, 14) + c0
                bk = jnp.bitwise_and(p, 16383)

                @pl.when((jk >= _TAIL0) & (g * 16 + k < cnt))
                def _():
                    pltpu.async_copy(tail_hbm.at[pl.ds(jk - _TAIL0, 1)],
                                     rowgrp_v.at[pl.ds(0, 1)], row_sem).wait()
                    pltpu.async_copy(rowgrp_v.at[pl.ds(0, 1)],
                                     out_hbm.at[pl.ds(bk, 1)],
                                     row_sem).wait()
            return t

        lax.fori_loop(0, ngrp, tail_walk, jnp.int32(0))

    return ret


@functools.partial(
    pl.kernel,
    mesh=_mesh,
    compiler_params=pltpu.CompilerParams(needs_layout_passes=False),
    out_type=[
        jax.ShapeDtypeStruct((B, D), jnp.float32),
        jax.ShapeDtypeStruct((B, D), jnp.float32),
    ],
    scratch_types=[
        pltpu.VMEM((D, _CW), jnp.float32),
        pltpu.VMEM((D, _CW), jnp.float32),
        pltpu.VMEM((D, _CW), jnp.float32),
        pltpu.VMEM((D, _CW), jnp.float32),
        pltpu.VMEM((1024,), jnp.int32),
        pltpu.VMEM((B + 16,), jnp.int32),
        pltpu.VMEM((B + 16,), jnp.int32),
        pltpu.VMEM((B + 16,), jnp.int32),
        pltpu.VMEM((16,), jnp.int32),
        pltpu.VMEM((64, D), jnp.float32),
        pltpu.SemaphoreType.DMA,
        pltpu.SemaphoreType.DMA,
        pltpu.SemaphoreType.DMA,
        pltpu.SemaphoreType.DMA,
        pltpu.SemaphoreType.DMA,
        pltpu.SemaphoreType.DMA,
    ],
)
def _sc_gather(uidx_hbm, iidx_hbm, utabT_hbm, itabT_hbm, utail_hbm, itail_hbm,
               uout_hbm, iout_hbm,
               buf0, buf1, buf2, buf3, idx_chunk_v, match_u_v, match_i_v,
               m2_v, rbase_v, rowgrp_v,
               sem0, sem1, sem2, sem3, idx_sem, row_sem):
    wid = lax.axis_index("s") * _NC + lax.axis_index("c")
    c0 = wid * (_CPW * _CW)
    last = wid == _NW - 1
    nch = jnp.where(last, _CPW + 1, _CPW)
    c1 = jnp.where(last, V, c0 + _CPW * _CW)
    cnt_u = _scan_indices(idx_chunk_v, uidx_hbm, idx_sem, c0, c1,
                          match_u_v)

    def scan_item_slice(t, ci):
        return lax.cond(
            t < B // 1024,
            lambda ci: _scan_one_chunk(t, ci, idx_chunk_v, iidx_hbm, idx_sem,
                                       c0, c1, match_i_v),
            lambda ci: ci, ci)

    cnt_i = _gather_phase(utabT_hbm, utail_hbm, uout_hbm,
                          (buf0, buf1, buf2, buf3),
                          (sem0, sem1, sem2, sem3), row_sem,
                          match_u_v, m2_v,
                          rbase_v, rowgrp_v, c0, c1, nch, wid, cnt_u,
                          background=scan_item_slice)
    _gather_phase(itabT_hbm, itail_hbm, iout_hbm, (buf0, buf1, buf2, buf3),
                  (sem0, sem1, sem2, sem3), row_sem,
                  match_i_v, m2_v,
                  rbase_v, rowgrp_v, c0, c1, nch, wid, cnt_i)


_BLK = 8192


def _mlp_body(uvec_ref, ivec_ref, w1u_ref, w1i_ref, b1_ref, w2_ref, b2_ref,
              w3_ref, b3_ref, out_ref):
    h = jnp.dot(uvec_ref[...], w1u_ref[...], preferred_element_type=jnp.float32)
    h += jnp.dot(ivec_ref[...], w1i_ref[...], preferred_element_type=jnp.float32)
    h = jnp.maximum(h + b1_ref[...], 0.0)
    h = jnp.dot(h, w2_ref[...], preferred_element_type=jnp.float32)
    h = jnp.maximum(h + b2_ref[...], 0.0)
    out_ref[...] = (
        jnp.dot(h, w3_ref[...], preferred_element_type=jnp.float32)
        + b3_ref[...]
    )


def _mlp(uvec, ivec, W1u, W1i, b1, W2, b2, W3, b3):
    grid = (B // _BLK,)
    return pl.pallas_call(
        _mlp_body,
        grid=grid,
        in_specs=[
            pl.BlockSpec((_BLK, D), lambda i: (i, 0)),
            pl.BlockSpec((_BLK, D), lambda i: (i, 0)),
            pl.BlockSpec((D, H1), lambda i: (0, 0)),
            pl.BlockSpec((D, H1), lambda i: (0, 0)),
            pl.BlockSpec((1, H1), lambda i: (0, 0)),
            pl.BlockSpec((H1, H2), lambda i: (0, 0)),
            pl.BlockSpec((1, H2), lambda i: (0, 0)),
            pl.BlockSpec((H2, 1), lambda i: (0, 0)),
            pl.BlockSpec((1, 1), lambda i: (0, 0)),
        ],
        out_specs=pl.BlockSpec((_BLK, 1), lambda i: (i, 0)),
        out_shape=jax.ShapeDtypeStruct((B, 1), jnp.float32),
    )(uvec, ivec, W1u, W1i, b1, W2, b2, W3, b3)


def kernel(user_idx, item_idx, user_table, item_table, W1, b1, W2, b2, W3, b3):
    user_idx = user_idx.astype(jnp.int32)
    item_idx = item_idx.astype(jnp.int32)
    uvec, ivec = _sc_gather(
        user_idx, item_idx, user_table.T, item_table.T,
        user_table[_TAIL0:], item_table[_TAIL0:])
    out = _mlp(
        uvec, ivec,
        W1[:D], W1[D:], b1.reshape(1, H1),
        W2, b2.reshape(1, H2),
        W3, b3.reshape(1, 1),
    )
    return jnp.squeeze(out, axis=-1)
